# baseline (speedup 1.0000x reference)
import jax
import jax.numpy as jnp
from jax import lax
from jax.experimental import pallas as pl
from jax.experimental.pallas import tpu as pltpu
from jax.experimental.pallas import tpu_sc as plsc

B = 1024
CTX_DIM = 512
N_CLS_CTX = 4
PRE = 5
SUF = 68
TOK = PRE + N_CLS_CTX + SUF

D_CLS = N_CLS_CTX * CTX_DIM
D_PRE = PRE * CTX_DIM
D_SUF = SUF * CTX_DIM
D_OUT = TOK * CTX_DIM

NC = 2
NS = 16
NW = NC * NS
BPW = B // NW
SREP = 16
FPS = BPW // NS


def _sc_body(label_hbm, table_hbm, pre_hbm, suf_hbm, out_hbm,
             idx_v, cls_v, pre_rep, suf_rep, gsem, osem):
  cid = lax.axis_index("c")
  sid = lax.axis_index("s")
  wid = sid * NC + cid
  base = wid * BPW

  pltpu.sync_copy(label_hbm.at[pl.ds(base, BPW)], idx_v)
  gather = pltpu.make_async_copy(table_hbm.at[idx_v], cls_v, gsem)
  gather.start()

  for k in range(FPS):
    r = sid * FPS + k
    pltpu.sync_copy(pre_hbm, pre_rep.at[r])
  pltpu.sync_copy(suf_hbm, suf_rep.at[sid])
  plsc.subcore_barrier()

  rows = pl.ds(base, BPW)
  copies = [pltpu.make_async_copy(
      pre_rep, out_hbm.at[rows, pl.ds(0, D_PRE)], osem)]
  for k in range(BPW // SREP):
    copies.append(pltpu.make_async_copy(
        suf_rep,
        out_hbm.at[pl.ds(base + k * SREP, SREP), pl.ds(D_PRE + D_CLS, D_SUF)],
        osem))
  for c in copies:
    c.start()
  gather.wait()
  cls_out = pltpu.make_async_copy(
      cls_v, out_hbm.at[rows, pl.ds(D_PRE, D_CLS)], osem)
  cls_out.start()
  copies.append(cls_out)
  for c in copies:
    c.wait()


@jax.jit
def kernel(label, clsctx, token_prefix, token_suffix):
  table = clsctx.reshape(clsctx.shape[0], D_CLS)
  pre = token_prefix.reshape(D_PRE)
  suf = token_suffix.reshape(D_SUF)
  idx = label.astype(jnp.int32)

  run = pl.kernel(
      _sc_body,
      out_type=jax.ShapeDtypeStruct((B, D_OUT), jnp.float32),
      mesh=plsc.VectorSubcoreMesh(core_axis_name="c", subcore_axis_name="s"),
      scratch_types=[
          pltpu.VMEM((BPW,), jnp.int32),
          pltpu.VMEM((BPW, D_CLS), jnp.float32),
          pltpu.VMEM_SHARED((BPW, D_PRE), jnp.float32),
          pltpu.VMEM_SHARED((SREP, D_SUF), jnp.float32),
          pltpu.SemaphoreType.DMA,
          pltpu.SemaphoreType.DMA,
      ],
  )
  out = run(idx, table, pre, suf)
  return out.reshape(B, TOK, CTX_DIM)


# speedup vs baseline: 4.6635x; 4.6635x over previous
import jax
import jax.numpy as jnp
from jax import lax
from jax.experimental import pallas as pl
from jax.experimental.pallas import tpu as pltpu
from jax.experimental.pallas import tpu_sc as plsc

B = 1024
CTX_DIM = 512
N_CLS_CTX = 4
PRE = 5
SUF = 68
TOK = PRE + N_CLS_CTX + SUF

NC = 2
NS = 16
NW = NC * NS
BPW = B // NW
G = 8
NCHUNK = BPW // G

BK = 16


def _sc_gather_body(label_hbm, table_hbm, out_hbm, idx_v, buf_v, gsem, osem):
  cid = lax.axis_index("c")
  sid = lax.axis_index("s")
  wid = sid * NC + cid
  base = wid * BPW

  pltpu.sync_copy(label_hbm.at[pl.ds(base, BPW)], idx_v)
  outs = [None] * NCHUNK
  for c in range(NCHUNK):
    b = c % 2
    if c >= 2:
      outs[c - 2].wait()
    gat = pltpu.make_async_copy(
        table_hbm.at[idx_v.at[pl.ds(c * G, G)]], buf_v.at[b], gsem[b])
    gat.start()
    gat.wait()
    outs[c] = pltpu.make_async_copy(
        buf_v.at[b], out_hbm.at[pl.ds(base + c * G, G)], osem[b])
    outs[c].start()
  outs[NCHUNK - 2].wait()
  outs[NCHUNK - 1].wait()


def _tc_assemble_body(pre_ref, suf_ref, cls_ref, out_ref):
  out_ref[:, 0:PRE, :] = jnp.broadcast_to(pre_ref[0], (BK, PRE, CTX_DIM))
  out_ref[:, PRE:PRE + N_CLS_CTX, :] = cls_ref[...]
  out_ref[:, PRE + N_CLS_CTX:TOK, :] = jnp.broadcast_to(
      suf_ref[0], (BK, SUF, CTX_DIM))


@jax.jit
def kernel(label, clsctx, token_prefix, token_suffix):
  idx = label.astype(jnp.int32)

  gather = pl.kernel(
      _sc_gather_body,
      out_type=jax.ShapeDtypeStruct((B, N_CLS_CTX, CTX_DIM), jnp.float32),
      mesh=plsc.VectorSubcoreMesh(core_axis_name="c", subcore_axis_name="s"),
      scratch_types=[
          pltpu.VMEM((BPW,), jnp.int32),
          pltpu.VMEM((2, G, N_CLS_CTX, CTX_DIM), jnp.float32),
          [pltpu.SemaphoreType.DMA, pltpu.SemaphoreType.DMA],
          [pltpu.SemaphoreType.DMA, pltpu.SemaphoreType.DMA],
      ],
  )
  cls = gather(idx, clsctx)

  assemble = pl.pallas_call(
      _tc_assemble_body,
      grid=(B // BK,),
      in_specs=[
          pl.BlockSpec((1, PRE, CTX_DIM), lambda i: (0, 0, 0)),
          pl.BlockSpec((1, SUF, CTX_DIM), lambda i: (0, 0, 0)),
          pl.BlockSpec((BK, N_CLS_CTX, CTX_DIM), lambda i: (i, 0, 0)),
      ],
      out_specs=pl.BlockSpec((BK, TOK, CTX_DIM), lambda i: (i, 0, 0)),
      out_shape=jax.ShapeDtypeStruct((B, TOK, CTX_DIM), jnp.float32),
  )
  return assemble(token_prefix, token_suffix, cls)


# speedup vs baseline: 4.7184x; 1.0118x over previous
import jax
import jax.numpy as jnp
from jax import lax
from jax.experimental import pallas as pl
from jax.experimental.pallas import tpu as pltpu
from jax.experimental.pallas import tpu_sc as plsc

B = 1024
CTX_DIM = 512
N_CLS_CTX = 4
PRE = 5
SUF = 68
TOK = PRE + N_CLS_CTX + SUF

HEAD = 16
TAIL = TOK - HEAD
SHIFT = HEAD - PRE - N_CLS_CTX

NC = 2
NS = 16
NW = NC * NS
BPW = B // NW
G = 8
NCHUNK = BPW // G
SREP = 8
LPC = CTX_DIM // 16


def _copy_rows(dst, dst_t0, src, src_t0, n_tok):
  def body(i, carry):
    t = i // LPC
    c = (i % LPC) * 16
    dst[dst_t0 + t, pl.ds(c, 16)] = src[src_t0 + t, pl.ds(c, 16)]
    return carry

  lax.fori_loop(0, n_tok * LPC, body, 0)


def _sc_body(label_hbm, table_hbm, pre_hbm, suf_hbm, out_hbm,
             idx_v, cls_v, pre_v, suf_v, tail_v, head_v,
             gsem, hsem, tsem):
  cid = lax.axis_index("c")
  sid = lax.axis_index("s")
  wid = sid * NC + cid
  base = wid * BPW

  pltpu.sync_copy(label_hbm.at[pl.ds(base, BPW)], idx_v)
  pltpu.sync_copy(pre_hbm.at[0], pre_v)
  pltpu.sync_copy(suf_hbm.at[0], suf_v)

  _copy_rows(tail_v, 0, suf_v, SHIFT, TAIL)

  for b in range(2):
    _copy_rows(head_v.at[b], 0, pre_v, 0, PRE)
    _copy_rows(head_v.at[b], PRE + N_CLS_CTX, suf_v, 0, SHIFT)

  prev = [None, None]
  tails = []
  for c in range(NCHUNK):
    gat = pltpu.make_async_copy(
        table_hbm.at[idx_v.at[pl.ds(c * G, G)]], cls_v, gsem)
    gat.start()
    gat.wait()
    for g in range(G):
      j = c * G + g
      rb = j % 2
      if prev[rb] is not None:
        prev[rb].wait()
      _copy_rows(head_v.at[rb], PRE, cls_v.at[g], 0, N_CLS_CTX)
      dma = pltpu.make_async_copy(
          head_v.at[rb], out_hbm.at[base + j, pl.ds(0, HEAD), :], hsem[rb])
      dma.start()
      prev[rb] = dma
      tdma = pltpu.make_async_copy(
          tail_v, out_hbm.at[base + j, pl.ds(HEAD, TAIL), :], tsem)
      tdma.start()
      tails.append(tdma)
      if len(tails) > 8:
        tails.pop(0).wait()
  prev[0].wait()
  prev[1].wait()
  for t in tails:
    t.wait()


@jax.jit
def kernel(label, clsctx, token_prefix, token_suffix):
  idx = label.astype(jnp.int32)

  run = pl.kernel(
      _sc_body,
      out_type=jax.ShapeDtypeStruct((B, TOK, CTX_DIM), jnp.float32),
      mesh=plsc.VectorSubcoreMesh(core_axis_name="c", subcore_axis_name="s"),
      scratch_types=[
          pltpu.VMEM((BPW,), jnp.int32),
          pltpu.VMEM((G, N_CLS_CTX, CTX_DIM), jnp.float32),
          pltpu.VMEM((PRE, CTX_DIM), jnp.float32),
          pltpu.VMEM((SUF, CTX_DIM), jnp.float32),
          pltpu.VMEM((TAIL, CTX_DIM), jnp.float32),
          pltpu.VMEM((2, HEAD, CTX_DIM), jnp.float32),
          pltpu.SemaphoreType.DMA,
          [pltpu.SemaphoreType.DMA, pltpu.SemaphoreType.DMA],
          pltpu.SemaphoreType.DMA,
      ],
  )
  return run(idx, clsctx, token_prefix, token_suffix)
